# baseline (speedup 1.0000x reference)
import functools

import jax
import jax.numpy as jnp
from jax import lax
from jax.experimental import pallas as pl
from jax.experimental.pallas import tpu as pltpu
from jax.experimental.pallas import tpu_sc as plsc

VOCAB = 1000000
DIM = 64
BATCH = 16384

_INFO = plsc.get_sparse_core_info()
_NC = _INFO.num_cores
_NS = _INFO.num_subcores
_NW = _NC * _NS
_L = _INFO.num_lanes

_ROWS_PER_W = BATCH // _NW
_CHUNK = 128
_NCHUNK = _ROWS_PER_W // _CHUNK


def _sc_kernel(cw_hbm, xw_hbm, ctab_hbm, xtab_hbm, out_hbm,
               cidx_v, xidx_v, crows_v, xrows_v, tbuf_v, out_v, sem):
    wid = lax.axis_index("s") * _NC + lax.axis_index("c")
    base = wid * _ROWS_PER_W

    pltpu.sync_copy(cw_hbm.at[wid], cidx_v)
    pltpu.sync_copy(xw_hbm.at[wid], xidx_v)

    copies = []
    for j in range(_NCHUNK):
        copies.append(pltpu.async_copy(
            ctab_hbm.at[cidx_v.at[j]], crows_v.at[pl.ds(j * _CHUNK, _CHUNK)],
            sem))
        copies.append(pltpu.async_copy(
            xtab_hbm.at[xidx_v.at[j]], xrows_v.at[pl.ds(j * _CHUNK, _CHUNK)],
            sem))
    for c in copies:
        c.wait()

    tidx = (_L + 1) * lax.iota(jnp.int32, _L)

    def body(g, carry):
        r0 = g * _L
        for i in range(_L):
            r = r0 + i
            acc = None
            for j in range(DIM // _L):
                cv = crows_v[r, pl.ds(j * _L, _L)]
                xv = xrows_v[r, pl.ds(j * _L, _L)]
                t = cv * xv
                acc = t if acc is None else acc + t
            tbuf_v[pl.ds(i * (_L + 1), _L)] = acc
        tot = jnp.zeros((_L,), jnp.float32)
        for j in range(_L):
            tot = tot + plsc.load_gather(tbuf_v, [tidx + j])
        out_v[pl.ds(r0, _L)] = tot
        return carry

    lax.fori_loop(0, _ROWS_PER_W // _L, body, 0, unroll=False)

    pltpu.sync_copy(out_v, out_hbm.at[pl.ds(base, _ROWS_PER_W)])


@jax.jit
def kernel(center_words, context_words, center_table, context_table):
    cw = center_words.astype(jnp.int32).reshape(_NW, _NCHUNK, _CHUNK)
    xw = context_words.astype(jnp.int32).reshape(_NW, _NCHUNK, _CHUNK)
    mesh = plsc.VectorSubcoreMesh(core_axis_name="c", subcore_axis_name="s")
    run = pl.kernel(
        _sc_kernel,
        out_type=jax.ShapeDtypeStruct((BATCH,), jnp.float32),
        mesh=mesh,
        scratch_types=[
            pltpu.VMEM((_NCHUNK, _CHUNK), jnp.int32),
            pltpu.VMEM((_NCHUNK, _CHUNK), jnp.int32),
            pltpu.VMEM((_ROWS_PER_W, DIM), jnp.float32),
            pltpu.VMEM((_ROWS_PER_W, DIM), jnp.float32),
            pltpu.VMEM((_L * (_L + 1),), jnp.float32),
            pltpu.VMEM((_ROWS_PER_W,), jnp.float32),
            pltpu.SemaphoreType.DMA,
        ],
        compiler_params=pltpu.CompilerParams(
            needs_layout_passes=False, use_tc_tiling_on_sc=False),
    )
    return run(cw, xw, center_table, context_table)
